# baseline (speedup 1.0000x reference)
import functools

import jax
import jax.numpy as jnp
from jax import lax
from jax.experimental import pallas as pl
from jax.experimental.pallas import tpu as pltpu
from jax.experimental.pallas import tpu_sc as plsc

DIM = 64
L = 16
SEQ = 50
BPC = 8
CH = SEQ * BPC
NIDX = 4
NW = 32
ROWS = 4096 * SEQ
RPW = ROWS // NW
NCH = RPW // CH
EPS = 1e-6

_DNUMS = lax.GatherDimensionNumbers(
    offset_dims=(), collapsed_slice_dims=(0,), start_index_map=(0,))


def _bcast_lanes(x, perms):
    for p in perms:
        x = x + lax.gather(x, p, _DNUMS, (1,),
                           mode=lax.GatherScatterMode.PROMISE_IN_BOUNDS)
    return x


def _ln_row(rows_v, out_v, bat, s, a2k, b2k, perms):
    row = bat * SEQ + s
    v = [rows_v[row, pl.ds(k * L, L)] for k in range(DIM // L)]
    sm = (v[0] + v[1]) + (v[2] + v[3])
    q = (v[0] * v[0] + v[1] * v[1]) + (v[2] * v[2] + v[3] * v[3])
    sm = _bcast_lanes(sm, perms)
    q = _bcast_lanes(q, perms)
    mean = sm * (1.0 / DIM)
    var = (q - sm * mean) * (1.0 / (DIM - 1))
    var = jnp.maximum(var, 0.0)
    y = plsc.bitcast(
        jnp.int32(0x5F3759DF) - (plsc.bitcast(var, jnp.int32) >> 1),
        jnp.float32,
    )
    for _ in range(3):
        y = y * (1.5 - 0.5 * var * y * y)
    inv = 1.0 / (var * y + EPS)
    for k in range(DIM // L):
        out_v[bat, s, pl.ds(k * L, L)] = (v[k] - mean) * inv * a2k[k] + b2k[k]


def _body(inp_hbm, table_hbm, a2_hbm, b2_hbm, out_hbm,
          idx_v, rows_v, out_v, a2_v, b2_v, gsem0, gsem1, osem0, osem1):
    wid = lax.axis_index("s") * 2 + lax.axis_index("c")
    pltpu.sync_copy(inp_hbm.at[wid], idx_v)
    pltpu.sync_copy(a2_hbm, a2_v)
    pltpu.sync_copy(b2_hbm, b2_v)
    a2k = [a2_v[pl.ds(k * L, L)] for k in range(DIM // L)]
    b2k = [b2_v[pl.ds(k * L, L)] for k in range(DIM // L)]
    iota = jnp.arange(L, dtype=jnp.int32)
    perms = [((iota ^ (1 << b))[:, None]) for b in range(4)]
    gsem = [gsem0, gsem1]
    osem = [osem0, osem1]

    def start_gather(c, b):
        for j in range(NIDX):
            pltpu.async_copy(
                table_hbm.at[idx_v.at[c, j]],
                rows_v.at[b, pl.ds(j * (CH // NIDX), CH // NIDX)], gsem[b])

    def wait_gather(c, b):
        for j in range(NIDX):
            pltpu.make_async_copy(
                table_hbm.at[idx_v.at[c, j]],
                rows_v.at[b, pl.ds(j * (CH // NIDX), CH // NIDX)],
                gsem[b]).wait()

    def compute(rows_ref, out_ref):
        def batch(bat, inner):
            for s in range(SEQ):
                _ln_row(rows_ref, out_ref, bat, s, a2k, b2k, perms)
            return inner

        lax.fori_loop(0, BPC, batch, 0)

    for b in range(2):
        start_gather(b, b)

    def chunk_pair(cc, carry):
        for b in range(2):
            c = 2 * cc + b
            bbase = pl.multiple_of(wid * (RPW // SEQ) + c * BPC, BPC)
            wait_gather(c, b)

            @pl.when(cc > 0)
            def _drain():
                pltpu.make_async_copy(
                    out_v.at[b], out_hbm.at[pl.ds(bbase, BPC)],
                    osem[b]).wait()

            compute(rows_v.at[b], out_v.at[b])
            pltpu.async_copy(
                out_v.at[b], out_hbm.at[pl.ds(bbase, BPC)], osem[b])

            @pl.when(cc < NCH // 2 - 1)
            def _prefetch():
                start_gather(c + 2, b)
        return carry

    lax.fori_loop(0, NCH // 2, chunk_pair, 0)
    for b in range(2):
        pltpu.make_async_copy(
            out_v.at[b],
            out_hbm.at[pl.ds(wid * (RPW // SEQ), BPC)], osem[b]).wait()


def kernel(inp, table, a_2, b_2):
    b, s = inp.shape
    inp_r = inp.reshape(NW, NCH, NIDX, CH // NIDX)
    mesh = plsc.VectorSubcoreMesh(core_axis_name="c", subcore_axis_name="s")
    run = functools.partial(
        pl.kernel,
        out_type=jax.ShapeDtypeStruct((b, s, DIM), jnp.float32),
        mesh=mesh,
        compiler_params=pltpu.CompilerParams(
            needs_layout_passes=False, use_tc_tiling_on_sc=False),
        scratch_types=[
            pltpu.VMEM((NCH, NIDX, CH // NIDX), jnp.int32),
            pltpu.VMEM((2, CH, DIM), jnp.float32),
            pltpu.VMEM((2, BPC, SEQ, DIM), jnp.float32),
            pltpu.VMEM((DIM,), jnp.float32),
            pltpu.VMEM((DIM,), jnp.float32),
            pltpu.SemaphoreType.DMA,
            pltpu.SemaphoreType.DMA,
            pltpu.SemaphoreType.DMA,
            pltpu.SemaphoreType.DMA,
        ],
    )(_body)
    return run(inp_r, table, a_2, b_2)


# speedup vs baseline: 1.0800x; 1.0800x over previous
import functools

import jax
import jax.numpy as jnp
from jax import lax
from jax.experimental import pallas as pl
from jax.experimental.pallas import tpu as pltpu
from jax.experimental.pallas import tpu_sc as plsc

DIM = 64
L = 16
CH = 128
NW = 32
ROWS = 4096 * 50
RPW = ROWS // NW
NCH = RPW // CH
GRP = 16
EPS = 1e-6

_DNUMS = lax.GatherDimensionNumbers(
    offset_dims=(), collapsed_slice_dims=(0,), start_index_map=(0,))


def _bcast_lanes(x, perms):
    for p in perms:
        x = x + lax.gather(x, p, _DNUMS, (1,),
                           mode=lax.GatherScatterMode.PROMISE_IN_BOUNDS)
    return x


def _ln_row(rows_v, out_v, row, a2k, b2k, perms):
    v = [rows_v[row, pl.ds(k * L, L)] for k in range(DIM // L)]
    s = (v[0] + v[1]) + (v[2] + v[3])
    q = (v[0] * v[0] + v[1] * v[1]) + (v[2] * v[2] + v[3] * v[3])
    s = _bcast_lanes(s, perms)
    q = _bcast_lanes(q, perms)
    mean = s * (1.0 / DIM)
    var = (q - s * mean) * (1.0 / (DIM - 1))
    var = jnp.maximum(var, 0.0)
    y = plsc.bitcast(
        jnp.int32(0x5F3759DF) - (plsc.bitcast(var, jnp.int32) >> 1),
        jnp.float32,
    )
    for _ in range(3):
        y = y * (1.5 - 0.5 * var * y * y)
    inv = 1.0 / (var * y + EPS)
    for k in range(DIM // L):
        out_v[row, pl.ds(k * L, L)] = (v[k] - mean) * inv * a2k[k] + b2k[k]


def _body(inp_hbm, table_hbm, a2_hbm, b2_hbm, out_hbm,
          idx_v, rows_v, out_v, a2_v, b2_v, gsem0, gsem1, osem0, osem1):
    wid = lax.axis_index("s") * 2 + lax.axis_index("c")
    pltpu.sync_copy(inp_hbm.at[wid], idx_v)
    pltpu.sync_copy(a2_hbm, a2_v)
    pltpu.sync_copy(b2_hbm, b2_v)
    a2k = [a2_v[pl.ds(k * L, L)] for k in range(DIM // L)]
    b2k = [b2_v[pl.ds(k * L, L)] for k in range(DIM // L)]
    iota = jnp.arange(L, dtype=jnp.int32)
    perms = [((iota ^ (1 << b))[:, None]) for b in range(4)]
    gsem = [gsem0, gsem1]
    osem = [osem0, osem1]

    def compute(rows_ref, out_ref):
        def group(g, inner):
            base = g * GRP
            for r in range(GRP):
                _ln_row(rows_ref, out_ref, base + r, a2k, b2k, perms)
            return inner

        lax.fori_loop(0, CH // GRP, group, 0)

    for b in range(2):
        pltpu.async_copy(table_hbm.at[idx_v.at[b]], rows_v.at[b], gsem[b])

    def chunk_pair(cc, carry):
        for b in range(2):
            c = 2 * cc + b
            obase = pl.multiple_of(wid * RPW + c * CH, CH)
            pltpu.make_async_copy(
                table_hbm.at[idx_v.at[c]], rows_v.at[b], gsem[b]).wait()

            @pl.when(cc > 0)
            def _drain():
                pltpu.make_async_copy(
                    out_v.at[b], out_hbm.at[pl.ds(obase, CH)], osem[b]).wait()

            compute(rows_v.at[b], out_v.at[b])
            pltpu.async_copy(
                out_v.at[b], out_hbm.at[pl.ds(obase, CH)], osem[b])

            @pl.when(cc < NCH // 2 - 1)
            def _prefetch():
                pltpu.async_copy(
                    table_hbm.at[idx_v.at[c + 2]], rows_v.at[b], gsem[b])
        return carry

    lax.fori_loop(0, NCH // 2, chunk_pair, 0)
    for b in range(2):
        pltpu.make_async_copy(
            out_v.at[b],
            out_hbm.at[pl.ds(wid * RPW, CH)], osem[b]).wait()


def kernel(inp, table, a_2, b_2):
    b, s = inp.shape
    inp_r = inp.reshape(NW, NCH, CH)
    mesh = plsc.VectorSubcoreMesh(core_axis_name="c", subcore_axis_name="s")
    run = functools.partial(
        pl.kernel,
        out_type=jax.ShapeDtypeStruct((ROWS, DIM), jnp.float32),
        mesh=mesh,
        compiler_params=pltpu.CompilerParams(
            needs_layout_passes=False, use_tc_tiling_on_sc=False),
        scratch_types=[
            pltpu.VMEM((NCH, CH), jnp.int32),
            pltpu.VMEM((2, CH, DIM), jnp.float32),
            pltpu.VMEM((2, CH, DIM), jnp.float32),
            pltpu.VMEM((DIM,), jnp.float32),
            pltpu.VMEM((DIM,), jnp.float32),
            pltpu.SemaphoreType.DMA,
            pltpu.SemaphoreType.DMA,
            pltpu.SemaphoreType.DMA,
            pltpu.SemaphoreType.DMA,
        ],
    )(_body)
    out = run(inp_r, table, a_2, b_2)
    return out.reshape(b, s, DIM)
